# baseline (speedup 1.0000x reference)
import jax
import jax.numpy as jnp
from jax.experimental import pallas as pl
from jax.experimental.pallas import tpu as pltpu

_BN_EPS = 1e-5
_BR = 200


def _fused_body(x_ref, w_ref, adj0_ref, adj1_ref, g_ref, b_ref, o_ref,
                sup_ref, acc_ref, s1_ref, s2_ref):
    i = pl.program_id(0)
    nb = pl.num_programs(0) - 1
    n = acc_ref.shape[0]

    @pl.when(i == 0)
    def _():
        sup_ref[...] = jnp.dot(x_ref[...], w_ref[...],
                               preferred_element_type=jnp.float32)

    @pl.when(i < nb)
    def _():
        sup = sup_ref[...]
        blk0 = jnp.dot(adj0_ref[...], sup, preferred_element_type=jnp.float32)
        blk1 = jnp.dot(adj1_ref[...], sup, preferred_element_type=jnp.float32)
        base = pl.multiple_of(i * (2 * _BR), 2 * _BR)
        acc_ref[pl.ds(base, _BR), :] = blk0
        acc_ref[pl.ds(base + _BR, _BR), :] = blk1
        csum = (jnp.sum(blk0, axis=0, keepdims=True)
                + jnp.sum(blk1, axis=0, keepdims=True))
        csq = (jnp.sum(blk0 * blk0, axis=0, keepdims=True)
               + jnp.sum(blk1 * blk1, axis=0, keepdims=True))

        @pl.when(i == 0)
        def _():
            s1_ref[...] = csum
            s2_ref[...] = csq

        @pl.when(i > 0)
        def _():
            s1_ref[...] += csum
            s2_ref[...] += csq

    @pl.when(i == nb)
    def _():
        mean = s1_ref[...] / n
        var = s2_ref[...] / n - mean * mean
        scale = g_ref[...] * jax.lax.rsqrt(var + _BN_EPS)
        shift = b_ref[...] - mean * scale
        o_ref[...] = jnp.tanh(acc_ref[...] * scale + shift)


def kernel(input, adj, W, bn_weight, bn_bias):
    n, din = input.shape
    dout = W.shape[1]
    nb = n // (2 * _BR)
    g = bn_weight.reshape(1, dout)
    b = bn_bias.reshape(1, dout)

    return pl.pallas_call(
        _fused_body,
        grid=(nb + 1,),
        in_specs=[
            pl.BlockSpec((n, din), lambda i: (0, 0)),
            pl.BlockSpec((din, dout), lambda i: (0, 0)),
            pl.BlockSpec((_BR, n),
                         lambda i: (jnp.minimum(2 * i, 2 * (n // (2 * _BR)) - 2), 0)),
            pl.BlockSpec((_BR, n),
                         lambda i: (jnp.minimum(2 * i + 1, 2 * (n // (2 * _BR)) - 1), 0)),
            pl.BlockSpec((1, dout), lambda i: (0, 0)),
            pl.BlockSpec((1, dout), lambda i: (0, 0)),
        ],
        out_specs=pl.BlockSpec((n, dout), lambda i: (0, 0)),
        out_shape=jax.ShapeDtypeStruct((n, dout), jnp.float32),
        scratch_shapes=[
            pltpu.VMEM((n, dout), jnp.float32),
            pltpu.VMEM((n, dout), jnp.float32),
            pltpu.VMEM((1, dout), jnp.float32),
            pltpu.VMEM((1, dout), jnp.float32),
        ],
    )(input, W, adj, adj, g, b)


# speedup vs baseline: 1.0071x; 1.0071x over previous
import jax
import jax.numpy as jnp
from jax.experimental import pallas as pl
from jax.experimental.pallas import tpu as pltpu

_BN_EPS = 1e-5
_BR = 400
_NT = 5


def _fused_body(x_ref, w_ref, adj_ref, g_ref, b_ref, o_ref,
                sup_ref, acc_ref, s1_ref, s2_ref):
    i = pl.program_id(0)
    nb = pl.num_programs(0) - _NT
    n = acc_ref.shape[0]
    nc = n // _NT

    @pl.when(i == 0)
    def _():
        sup_ref[...] = jnp.dot(x_ref[...], w_ref[...],
                               preferred_element_type=jnp.float32)

    @pl.when(i < nb)
    def _():
        blk = jnp.dot(adj_ref[...], sup_ref[...],
                      preferred_element_type=jnp.float32)
        base = pl.multiple_of(i * _BR, _BR)
        acc_ref[pl.ds(base, _BR), :] = blk
        csum = jnp.sum(blk, axis=0, keepdims=True)
        csq = jnp.sum(blk * blk, axis=0, keepdims=True)

        @pl.when(i == 0)
        def _():
            s1_ref[...] = csum
            s2_ref[...] = csq

        @pl.when(i > 0)
        def _():
            s1_ref[...] += csum
            s2_ref[...] += csq

    @pl.when(i >= nb)
    def _():
        mean = s1_ref[...] / n
        var = s2_ref[...] / n - mean * mean
        scale = g_ref[...] * jax.lax.rsqrt(var + _BN_EPS)
        shift = b_ref[...] - mean * scale
        base = pl.multiple_of((i - nb) * nc, nc)
        o_ref[...] = jnp.tanh(acc_ref[pl.ds(base, nc), :] * scale + shift)


def kernel(input, adj, W, bn_weight, bn_bias):
    n, din = input.shape
    dout = W.shape[1]
    nb = n // _BR
    g = bn_weight.reshape(1, dout)
    b = bn_bias.reshape(1, dout)

    return pl.pallas_call(
        _fused_body,
        grid=(nb + _NT,),
        in_specs=[
            pl.BlockSpec((n, din), lambda i: (0, 0)),
            pl.BlockSpec((din, dout), lambda i: (0, 0)),
            pl.BlockSpec((_BR, n),
                         lambda i: (jnp.minimum(i, n // _BR - 1), 0)),
            pl.BlockSpec((1, dout), lambda i: (0, 0)),
            pl.BlockSpec((1, dout), lambda i: (0, 0)),
        ],
        out_specs=pl.BlockSpec(
            (n // _NT, dout),
            lambda i: (jnp.maximum(i - (n // _BR), 0), 0)),
        out_shape=jax.ShapeDtypeStruct((n, dout), jnp.float32),
        scratch_shapes=[
            pltpu.VMEM((n, dout), jnp.float32),
            pltpu.VMEM((n, dout), jnp.float32),
            pltpu.VMEM((1, dout), jnp.float32),
            pltpu.VMEM((1, dout), jnp.float32),
        ],
    )(input, W, adj, g, b)


# speedup vs baseline: 1.0080x; 1.0009x over previous
import jax
import jax.numpy as jnp
from jax.experimental import pallas as pl
from jax.experimental.pallas import tpu as pltpu

_BN_EPS = 1e-5
_BR = 400
_NT = 5


def _fused_body(x_ref, w_ref, adj_ref, g_ref, b_ref, o_ref,
                sup_ref, acc_ref, s1_ref, s2_ref):
    i = pl.program_id(0)
    nb = pl.num_programs(0) - _NT
    n = acc_ref.shape[0]
    nc = n // _NT

    @pl.when(i == 0)
    def _():
        sup_ref[...] = jnp.dot(x_ref[...], w_ref[...],
                               preferred_element_type=jnp.float32
                               ).astype(jnp.bfloat16)

    @pl.when(i < nb)
    def _():
        blk = jnp.dot(adj_ref[...], sup_ref[...],
                      preferred_element_type=jnp.float32)
        base = pl.multiple_of(i * _BR, _BR)
        acc_ref[pl.ds(base, _BR), :] = blk
        csum = jnp.sum(blk, axis=0, keepdims=True)
        csq = jnp.sum(blk * blk, axis=0, keepdims=True)

        @pl.when(i == 0)
        def _():
            s1_ref[...] = csum
            s2_ref[...] = csq

        @pl.when(i > 0)
        def _():
            s1_ref[...] += csum
            s2_ref[...] += csq

    @pl.when(i >= nb)
    def _():
        mean = s1_ref[...] / n
        var = s2_ref[...] / n - mean * mean
        scale = g_ref[...] * jax.lax.rsqrt(var + _BN_EPS)
        shift = b_ref[...] - mean * scale
        base = pl.multiple_of((i - nb) * nc, nc)
        o_ref[...] = jnp.tanh(acc_ref[pl.ds(base, nc), :] * scale + shift)


def kernel(input, adj, W, bn_weight, bn_bias):
    n, din = input.shape
    dout = W.shape[1]
    nb = n // _BR
    g = bn_weight.reshape(1, dout)
    b = bn_bias.reshape(1, dout)

    return pl.pallas_call(
        _fused_body,
        grid=(nb + _NT,),
        in_specs=[
            pl.BlockSpec((n, din), lambda i: (0, 0)),
            pl.BlockSpec((din, dout), lambda i: (0, 0)),
            pl.BlockSpec((_BR, n),
                         lambda i: (jnp.minimum(i, n // _BR - 1), 0)),
            pl.BlockSpec((1, dout), lambda i: (0, 0)),
            pl.BlockSpec((1, dout), lambda i: (0, 0)),
        ],
        out_specs=pl.BlockSpec(
            (n // _NT, dout),
            lambda i: (jnp.maximum(i - (n // _BR), 0), 0)),
        out_shape=jax.ShapeDtypeStruct((n, dout), jnp.float32),
        scratch_shapes=[
            pltpu.VMEM((n, dout), jnp.bfloat16),
            pltpu.VMEM((n, dout), jnp.float32),
            pltpu.VMEM((1, dout), jnp.float32),
            pltpu.VMEM((1, dout), jnp.float32),
        ],
    )(input, W, adj, g, b)


# speedup vs baseline: 1.0151x; 1.0071x over previous
import jax
import jax.numpy as jnp
from jax.experimental import pallas as pl
from jax.experimental.pallas import tpu as pltpu

_BN_EPS = 1e-5
_BR = 400
_NT = 2


def _fused_body(x_ref, w_ref, adj_ref, g_ref, b_ref, o_ref,
                sup_ref, acc_ref, s1_ref, s2_ref):
    i = pl.program_id(0)
    nb = pl.num_programs(0) - _NT
    n = acc_ref.shape[0]
    nc = n // _NT

    @pl.when(i == 0)
    def _():
        sup_ref[...] = jnp.dot(x_ref[...], w_ref[...],
                               preferred_element_type=jnp.float32
                               ).astype(jnp.bfloat16)

    @pl.when(i < nb)
    def _():
        blk = jnp.dot(adj_ref[...], sup_ref[...],
                      preferred_element_type=jnp.float32)
        base = pl.multiple_of(i * _BR, _BR)
        acc_ref[pl.ds(base, _BR), :] = blk
        csum = jnp.sum(blk, axis=0, keepdims=True)
        csq = jnp.sum(blk * blk, axis=0, keepdims=True)

        @pl.when(i == 0)
        def _():
            s1_ref[...] = csum
            s2_ref[...] = csq

        @pl.when(i > 0)
        def _():
            s1_ref[...] += csum
            s2_ref[...] += csq

    @pl.when(i >= nb)
    def _():
        mean = s1_ref[...] / n
        var = s2_ref[...] / n - mean * mean
        scale = g_ref[...] * jax.lax.rsqrt(var + _BN_EPS)
        shift = b_ref[...] - mean * scale
        base = pl.multiple_of((i - nb) * nc, nc)
        o_ref[...] = jnp.tanh(acc_ref[pl.ds(base, nc), :] * scale + shift)


def kernel(input, adj, W, bn_weight, bn_bias):
    n, din = input.shape
    dout = W.shape[1]
    nb = n // _BR
    g = bn_weight.reshape(1, dout)
    b = bn_bias.reshape(1, dout)

    return pl.pallas_call(
        _fused_body,
        grid=(nb + _NT,),
        in_specs=[
            pl.BlockSpec((n, din), lambda i: (0, 0)),
            pl.BlockSpec((din, dout), lambda i: (0, 0)),
            pl.BlockSpec((_BR, n),
                         lambda i: (jnp.minimum(i, n // _BR - 1), 0)),
            pl.BlockSpec((1, dout), lambda i: (0, 0)),
            pl.BlockSpec((1, dout), lambda i: (0, 0)),
        ],
        out_specs=pl.BlockSpec(
            (n // _NT, dout),
            lambda i: (jnp.maximum(i - (n // _BR), 0), 0)),
        out_shape=jax.ShapeDtypeStruct((n, dout), jnp.float32),
        scratch_shapes=[
            pltpu.VMEM((n, dout), jnp.bfloat16),
            pltpu.VMEM((n, dout), jnp.float32),
            pltpu.VMEM((1, dout), jnp.float32),
            pltpu.VMEM((1, dout), jnp.float32),
        ],
    )(input, W, adj, g, b)
